# initial kernel scaffold (unmeasured)
import jax
import jax.numpy as jnp
from jax import lax
from jax.experimental import pallas as pl
from jax.experimental.pallas import tpu as pltpu

N_DEV = 32
B, SQ, D = 4, 256, 1024
HQ, DH = 8, 128
ROWS = B * SQ
CHUNK = ROWS // N_DEV
SCALE = 0.08838834764831843


def _body(x_ref, wq_ref, wo_ref, wk_ref, wv_ref, out_ref,
          recv_rs, rs_send, rs_recv, ag_send, ag_recv):
    my = lax.axis_index("i")
    left = lax.rem(my + N_DEV - 1, N_DEV)
    right = lax.rem(my + 1, N_DEV)

    xf = x_ref[:, :]
    q = jnp.dot(xf, wq_ref[:, :], preferred_element_type=jnp.float32
                ).astype(jnp.bfloat16)
    k = jnp.dot(xf, wk_ref[:, :], preferred_element_type=jnp.float32
                ).astype(jnp.bfloat16)
    v = jnp.dot(xf, wv_ref[:, :], preferred_element_type=jnp.float32
                ).astype(jnp.bfloat16)

    for b in range(B):
        r0 = b * SQ
        acc = jnp.zeros((SQ, D), jnp.float32)
        for h in range(HQ):
            c0 = h * DH
            qb = q[r0:r0 + SQ, c0:c0 + DH]
            kb = k[r0:r0 + SQ, c0:c0 + DH]
            vb = v[r0:r0 + SQ, c0:c0 + DH]
            s = lax.dot_general(
                qb, kb, (((1,), (1,)), ((), ())),
                preferred_element_type=jnp.float32) * SCALE
            m = jnp.max(s, axis=1, keepdims=True)
            p = jnp.exp(s - m)
            l = jnp.sum(p, axis=1, keepdims=True)
            pb = (p / l).astype(jnp.bfloat16)
            ob = jnp.dot(pb, vb, preferred_element_type=jnp.float32
                         ).astype(jnp.bfloat16)
            acc = acc + jnp.dot(ob, wo_ref[c0:c0 + DH, :],
                                preferred_element_type=jnp.float32)
        out_ref[r0:r0 + SQ, :] = acc

    barrier = pltpu.get_barrier_semaphore()
    for nbr in (left, right):
        pl.semaphore_signal(barrier, inc=1, device_id=(nbr,),
                            device_id_type=pl.DeviceIdType.MESH)
    pl.semaphore_wait(barrier, 2)

    for h in range(N_DEV - 1):
        s_c = lax.rem(my - h + N_DEV, N_DEV)
        if h > 0:
            out_ref[pl.ds(s_c * CHUNK, CHUNK), :] = (
                out_ref[pl.ds(s_c * CHUNK, CHUNK), :] + recv_rs[h - 1])
        rdma = pltpu.make_async_remote_copy(
            src_ref=out_ref.at[pl.ds(s_c * CHUNK, CHUNK), :],
            dst_ref=recv_rs.at[h],
            send_sem=rs_send.at[h],
            recv_sem=rs_recv.at[h],
            device_id=(right,),
            device_id_type=pl.DeviceIdType.MESH,
        )
        rdma.start()
        rdma.wait()
    f_c = lax.rem(my + 1, N_DEV)
    out_ref[pl.ds(f_c * CHUNK, CHUNK), :] = (
        out_ref[pl.ds(f_c * CHUNK, CHUNK), :] + recv_rs[N_DEV - 2])

    for h in range(N_DEV - 1):
        s_c = lax.rem(my + 1 - h + N_DEV, N_DEV)
        r_c = lax.rem(my - h + N_DEV, N_DEV)
        rdma = pltpu.make_async_remote_copy(
            src_ref=out_ref.at[pl.ds(s_c * CHUNK, CHUNK), :],
            dst_ref=out_ref.at[pl.ds(r_c * CHUNK, CHUNK), :],
            send_sem=ag_send.at[h],
            recv_sem=ag_recv.at[h],
            device_id=(right,),
            device_id_type=pl.DeviceIdType.MESH,
        )
        rdma.start()
        rdma.wait()


def kernel(x, Wq, Wo, Wk, Wv):
    xf = x.reshape(ROWS, D).astype(jnp.bfloat16)
    wq = Wq.astype(jnp.bfloat16)
    wk = Wk.astype(jnp.bfloat16)
    wv = Wv.astype(jnp.bfloat16)
    wo = Wo.astype(jnp.bfloat16)

    out = pl.pallas_call(
        _body,
        out_shape=jax.ShapeDtypeStruct((ROWS, D), jnp.float32),
        in_specs=[pl.BlockSpec(memory_space=pltpu.VMEM)] * 5,
        out_specs=pl.BlockSpec(memory_space=pltpu.VMEM),
        scratch_shapes=[
            pltpu.VMEM((N_DEV - 1, CHUNK, D), jnp.float32),
            pltpu.SemaphoreType.DMA((N_DEV - 1,)),
            pltpu.SemaphoreType.DMA((N_DEV - 1,)),
            pltpu.SemaphoreType.DMA((N_DEV - 1,)),
            pltpu.SemaphoreType.DMA((N_DEV - 1,)),
        ],
        compiler_params=pltpu.CompilerParams(collective_id=0),
    )(xf, wq, wo, wk, wv)
    return out.reshape(B, SQ, D)


# baseline (device time: 244108 ns/iter reference)
import jax
import jax.numpy as jnp
from jax import lax
from jax.experimental import pallas as pl
from jax.experimental.pallas import tpu as pltpu

N_DEV = 32
B, SQ, D = 4, 256, 1024
HQ, DH = 8, 128
ROWS = B * SQ
CHUNK = ROWS // N_DEV
SCALE = 0.08838834764831843


def _body(x_ref, wq_ref, wo_ref, wk_ref, wv_ref, out_ref,
          recv_rs, rs_send, rs_recv, ag_send, ag_recv):
    my = lax.axis_index("i")
    left = lax.rem(my + N_DEV - 1, N_DEV)
    right = lax.rem(my + 1, N_DEV)

    xf = x_ref[:, :]
    q = jnp.dot(xf, wq_ref[:, :], preferred_element_type=jnp.float32
                ).astype(jnp.bfloat16)
    k = jnp.dot(xf, wk_ref[:, :], preferred_element_type=jnp.float32
                ).astype(jnp.bfloat16)
    v = jnp.dot(xf, wv_ref[:, :], preferred_element_type=jnp.float32
                ).astype(jnp.bfloat16)

    for b in range(B):
        r0 = b * SQ
        acc = jnp.zeros((SQ, D), jnp.float32)
        for h in range(HQ):
            c0 = h * DH
            qb = q[r0:r0 + SQ, c0:c0 + DH]
            kb = k[r0:r0 + SQ, c0:c0 + DH]
            vb = v[r0:r0 + SQ, c0:c0 + DH]
            s = lax.dot_general(
                qb, kb, (((1,), (1,)), ((), ())),
                preferred_element_type=jnp.float32) * SCALE
            m = jnp.max(s, axis=1, keepdims=True)
            p = jnp.exp(s - m)
            l = jnp.sum(p, axis=1, keepdims=True)
            pb = (p / l).astype(jnp.bfloat16)
            ob = jnp.dot(pb, vb, preferred_element_type=jnp.float32
                         ).astype(jnp.bfloat16)
            acc = acc + jnp.dot(ob, wo_ref[c0:c0 + DH, :],
                                preferred_element_type=jnp.float32)
        out_ref[r0:r0 + SQ, :] = acc

    barrier = pltpu.get_barrier_semaphore()
    for nbr in (left, right):
        pl.semaphore_signal(barrier, inc=1, device_id=(nbr,),
                            device_id_type=pl.DeviceIdType.MESH)
    pl.semaphore_wait(barrier, 2)

    for h in range(N_DEV - 1):
        s_c = lax.rem(my - h + N_DEV, N_DEV)
        if h > 0:
            out_ref[pl.ds(s_c * CHUNK, CHUNK), :] = (
                out_ref[pl.ds(s_c * CHUNK, CHUNK), :] + recv_rs[h - 1])
        rdma = pltpu.make_async_remote_copy(
            src_ref=out_ref.at[pl.ds(s_c * CHUNK, CHUNK), :],
            dst_ref=recv_rs.at[h],
            send_sem=rs_send.at[h],
            recv_sem=rs_recv.at[h],
            device_id=(right,),
            device_id_type=pl.DeviceIdType.MESH,
        )
        rdma.start()
        rdma.wait()
    f_c = lax.rem(my + 1, N_DEV)
    out_ref[pl.ds(f_c * CHUNK, CHUNK), :] = (
        out_ref[pl.ds(f_c * CHUNK, CHUNK), :] + recv_rs[N_DEV - 2])

    for h in range(N_DEV - 1):
        s_c = lax.rem(my + 1 - h + N_DEV, N_DEV)
        rdma = pltpu.make_async_remote_copy(
            src_ref=out_ref.at[pl.ds(s_c * CHUNK, CHUNK), :],
            dst_ref=out_ref.at[pl.ds(s_c * CHUNK, CHUNK), :],
            send_sem=ag_send.at[h],
            recv_sem=ag_recv.at[h],
            device_id=(right,),
            device_id_type=pl.DeviceIdType.MESH,
        )
        rdma.start()
        rdma.wait()


def kernel(x, Wq, Wo, Wk, Wv):
    xf = x.reshape(ROWS, D).astype(jnp.bfloat16)
    wq = Wq.astype(jnp.bfloat16)
    wk = Wk.astype(jnp.bfloat16)
    wv = Wv.astype(jnp.bfloat16)
    wo = Wo.astype(jnp.bfloat16)

    out = pl.pallas_call(
        _body,
        out_shape=jax.ShapeDtypeStruct((ROWS, D), jnp.float32),
        in_specs=[pl.BlockSpec(memory_space=pltpu.VMEM)] * 5,
        out_specs=pl.BlockSpec(memory_space=pltpu.VMEM),
        scratch_shapes=[
            pltpu.VMEM((N_DEV - 1, CHUNK, D), jnp.float32),
            pltpu.SemaphoreType.DMA((N_DEV - 1,)),
            pltpu.SemaphoreType.DMA((N_DEV - 1,)),
            pltpu.SemaphoreType.DMA((N_DEV - 1,)),
            pltpu.SemaphoreType.DMA((N_DEV - 1,)),
        ],
        compiler_params=pltpu.CompilerParams(collective_id=0),
    )(xf, wq, wo, wk, wv)
    return out.reshape(B, SQ, D)


# device time: 107823 ns/iter; 2.2640x vs baseline; 2.2640x over previous
import functools

import jax
import jax.numpy as jnp
from jax import lax
from jax.experimental import pallas as pl
from jax.experimental.pallas import tpu as pltpu

N_DEV = 32
B, SQ, D = 4, 256, 1024
HQ, DH = 8, 128
ROWS = B * SQ
CHUNK = ROWS // N_DEV
SCALE = 0.08838834764831843

RS_STAGES = [(16, "x"), (8, "y1"), (4, "z1"), (2, "y2"), (1, "z2")]


def _rank_from_xyz(x, y, z):
    return z * 8 + y * 2 + (x ^ (y & 1))


def _decompose(my):
    z = my // 8
    p = lax.rem(my, 8)
    y = p // 2
    x = lax.rem(p, 2) ^ (y & 1)
    return x, y, z


def _partner(my, axis):
    x, y, z = _decompose(my)
    if axis == "x":
        x = 1 - x
    elif axis == "y1":
        y = y ^ 1
    elif axis == "y2":
        y = y ^ 2
    elif axis == "z1":
        z = z ^ 1
    elif axis == "z2":
        z = z ^ 2
    return _rank_from_xyz(x, y, z)


def _body(x_ref, wq_ref, wo_ref, wk_ref, wv_ref, out_ref, work,
          sb16, sb8, sb4, sb2, sb1, rb16, rb8, rb4, rb2, rb1,
          rs_ssem, rs_rsem, ag_ssem, ag_rsem):
    my = lax.axis_index("i")
    xb, yb, zb = _decompose(my)
    bits = {
        "x": xb,
        "y1": yb & 1,
        "z1": zb & 1,
        "y2": (yb >> 1) & 1,
        "z2": (zb >> 1) & 1,
    }
    c_my = (bits["x"] * 16 + bits["y1"] * 8 + bits["z1"] * 4
            + bits["y2"] * 2 + bits["z2"])
    partners = {ax: _partner(my, ax) for _, ax in RS_STAGES}
    sbufs = {16: sb16, 8: sb8, 4: sb4, 2: sb2, 1: sb1}
    rbufs = {16: rb16, 8: rb8, 4: rb4, 2: rb2, 1: rb1}

    xf = x_ref[:, :]
    q = jnp.dot(xf, wq_ref[:, :], preferred_element_type=jnp.float32
                ).astype(jnp.bfloat16)
    k = jnp.dot(xf, wk_ref[:, :], preferred_element_type=jnp.float32
                ).astype(jnp.bfloat16)
    v = jnp.dot(xf, wv_ref[:, :], preferred_element_type=jnp.float32
                ).astype(jnp.bfloat16)

    for b in range(B):
        r0 = b * SQ
        acc = jnp.zeros((SQ, D), jnp.float32)
        for h in range(HQ):
            c0 = h * DH
            qb = q[r0:r0 + SQ, c0:c0 + DH]
            kb = k[r0:r0 + SQ, c0:c0 + DH]
            vb = v[r0:r0 + SQ, c0:c0 + DH]
            s = lax.dot_general(
                qb, kb, (((1,), (1,)), ((), ())),
                preferred_element_type=jnp.float32) * SCALE
            m = jnp.max(s, axis=1, keepdims=True)
            p = jnp.exp(s - m)
            l = jnp.sum(p, axis=1, keepdims=True)
            pb = (p / l).astype(jnp.bfloat16)
            ob = jnp.dot(pb, vb, preferred_element_type=jnp.float32
                         ).astype(jnp.bfloat16)
            acc = acc + jnp.dot(ob, wo_ref[c0:c0 + DH, :],
                                preferred_element_type=jnp.float32)
        work[r0:r0 + SQ, :] = acc

    barrier = pltpu.get_barrier_semaphore()
    for _, ax in RS_STAGES:
        pl.semaphore_signal(barrier, inc=1, device_id=(partners[ax],),
                            device_id_type=pl.DeviceIdType.MESH)
    pl.semaphore_wait(barrier, len(RS_STAGES))

    for s, (w, ax) in enumerate(RS_STAGES):
        beta = bits[ax]
        lo = (c_my // (2 * w)) * (2 * w)
        send_lo = (lo + (1 - beta) * w) * CHUNK
        keep_lo = (lo + beta * w) * CHUNK
        L = w * CHUNK
        sbuf, rbuf = sbufs[w], rbufs[w]
        sbuf[:, :] = work[pl.ds(send_lo, L), :].astype(jnp.bfloat16)
        rdma = pltpu.make_async_remote_copy(
            src_ref=sbuf,
            dst_ref=rbuf,
            send_sem=rs_ssem.at[s],
            recv_sem=rs_rsem.at[s],
            device_id=(partners[ax],),
            device_id_type=pl.DeviceIdType.MESH,
        )
        rdma.start()
        rdma.wait()
        work[pl.ds(keep_lo, L), :] = (
            work[pl.ds(keep_lo, L), :] + rbuf[:, :].astype(jnp.float32))

    out_ref[pl.ds(c_my * CHUNK, CHUNK), :] = (
        work[pl.ds(c_my * CHUNK, CHUNK), :].astype(jnp.bfloat16))

    for s, (w, ax) in enumerate(reversed(RS_STAGES)):
        cur_lo = (c_my // w) * w * CHUNK
        L = w * CHUNK
        rdma = pltpu.make_async_remote_copy(
            src_ref=out_ref.at[pl.ds(cur_lo, L), :],
            dst_ref=out_ref.at[pl.ds(cur_lo, L), :],
            send_sem=ag_ssem.at[s],
            recv_sem=ag_rsem.at[s],
            device_id=(partners[ax],),
            device_id_type=pl.DeviceIdType.MESH,
        )
        rdma.start()
        rdma.wait()

    @functools.partial(pl.run_scoped, exit_sem=pltpu.SemaphoreType.REGULAR)
    def _(exit_sem):
        for _, ax in RS_STAGES:
            pl.semaphore_signal(exit_sem, inc=1, device_id=(partners[ax],),
                                device_id_type=pl.DeviceIdType.MESH)
        pl.semaphore_wait(exit_sem, len(RS_STAGES))


def kernel(x, Wq, Wo, Wk, Wv):
    xf = x.reshape(ROWS, D).astype(jnp.bfloat16)
    wq = Wq.astype(jnp.bfloat16)
    wk = Wk.astype(jnp.bfloat16)
    wv = Wv.astype(jnp.bfloat16)
    wo = Wo.astype(jnp.bfloat16)

    out = pl.pallas_call(
        _body,
        out_shape=jax.ShapeDtypeStruct((ROWS, D), jnp.bfloat16),
        in_specs=[pl.BlockSpec(memory_space=pltpu.VMEM)] * 5,
        out_specs=pl.BlockSpec(memory_space=pltpu.VMEM),
        scratch_shapes=(
            [pltpu.VMEM((ROWS, D), jnp.float32)]
            + [pltpu.VMEM((w * CHUNK, D), jnp.bfloat16)
               for w, _ in RS_STAGES]
            + [pltpu.VMEM((w * CHUNK, D), jnp.bfloat16)
               for w, _ in RS_STAGES]
            + [pltpu.SemaphoreType.DMA((len(RS_STAGES),))
               , pltpu.SemaphoreType.DMA((len(RS_STAGES),))
               , pltpu.SemaphoreType.DMA((len(RS_STAGES),))
               , pltpu.SemaphoreType.DMA((len(RS_STAGES),))]
        ),
        compiler_params=pltpu.CompilerParams(collective_id=0),
    )(xf, wq, wo, wk, wv)
    return out.astype(jnp.float32).reshape(B, SQ, D)


# device time: 87663 ns/iter; 2.7846x vs baseline; 1.2300x over previous
import functools

import jax
import jax.numpy as jnp
from jax import lax
from jax.experimental import pallas as pl
from jax.experimental.pallas import tpu as pltpu

N_DEV = 32
B, SQ, D = 4, 256, 1024
HQ, DH = 8, 128
ROWS = B * SQ
CHUNK = ROWS // N_DEV
N_LANE = 4
CD = D // N_LANE
SCALE = 0.08838834764831843

AXES = ("x", "y1", "z1", "y2", "z2")
LANE_ORDERS = (
    ("x", "y1", "z1", "y2", "z2"),
    ("y1", "z1", "x", "z2", "y2"),
    ("z1", "z2", "y2", "x", "y1"),
    ("y2", "x", "z2", "y1", "z1"),
)
N_STAGE = 5


def _rank_from_xyz(x, y, z):
    return z * 8 + y * 2 + (x ^ (y & 1))


def _decompose(my):
    z = my // 8
    p = lax.rem(my, 8)
    y = p // 2
    x = lax.rem(p, 2) ^ (y & 1)
    return x, y, z


def _partner(my, axis):
    x, y, z = _decompose(my)
    if axis == "x":
        x = 1 - x
    elif axis == "y1":
        y = y ^ 1
    elif axis == "y2":
        y = y ^ 2
    elif axis == "z1":
        z = z ^ 1
    elif axis == "z2":
        z = z ^ 2
    return _rank_from_xyz(x, y, z)


def _body(x_ref, wq_ref, wo_ref, wk_ref, wv_ref, out_ref, work, *scratch):
    sbufs = scratch[0:20]
    rbufs = scratch[20:40]
    rs_ssem, rs_rsem, ag_ssem, ag_rsem = scratch[40:44]

    my = lax.axis_index("i")
    xb, yb, zb = _decompose(my)
    bits = {
        "x": xb,
        "y1": yb & 1,
        "z1": zb & 1,
        "y2": (yb >> 1) & 1,
        "z2": (zb >> 1) & 1,
    }
    partners = {ax: _partner(my, ax) for ax in AXES}
    c_lane = [
        sum(bits[ax] * (16 >> k) for k, ax in enumerate(order))
        for order in LANE_ORDERS
    ]

    xf = x_ref[:, :]
    q = jnp.dot(xf, wq_ref[:, :], preferred_element_type=jnp.float32
                ).astype(jnp.bfloat16)
    k = jnp.dot(xf, wk_ref[:, :], preferred_element_type=jnp.float32
                ).astype(jnp.bfloat16)
    v = jnp.dot(xf, wv_ref[:, :], preferred_element_type=jnp.float32
                ).astype(jnp.bfloat16)

    for b in range(B):
        r0 = b * SQ
        acc = jnp.zeros((SQ, D), jnp.float32)
        for h in range(HQ):
            c0 = h * DH
            qb = q[r0:r0 + SQ, c0:c0 + DH]
            kb = k[r0:r0 + SQ, c0:c0 + DH]
            vb = v[r0:r0 + SQ, c0:c0 + DH]
            s = lax.dot_general(
                qb, kb, (((1,), (1,)), ((), ())),
                preferred_element_type=jnp.float32) * SCALE
            m = jnp.max(s, axis=1, keepdims=True)
            p = jnp.exp(s - m)
            l = jnp.sum(p, axis=1, keepdims=True)
            pb = (p / l).astype(jnp.bfloat16)
            ob = jnp.dot(pb, vb, preferred_element_type=jnp.float32
                         ).astype(jnp.bfloat16)
            acc = acc + jnp.dot(ob, wo_ref[c0:c0 + DH, :],
                                preferred_element_type=jnp.float32)
        work[r0:r0 + SQ, :] = acc

    barrier = pltpu.get_barrier_semaphore()
    for ax in AXES:
        pl.semaphore_signal(barrier, inc=1, device_id=(partners[ax],),
                            device_id_type=pl.DeviceIdType.MESH)
    pl.semaphore_wait(barrier, len(AXES))

    rdmas = [None] * N_LANE
    for s in range(N_STAGE):
        w = 16 >> s
        L = w * CHUNK
        for j, order in enumerate(LANE_ORDERS):
            ax = order[s]
            beta = bits[ax]
            lo = (c_lane[j] // (2 * w)) * (2 * w)
            send_lo = (lo + (1 - beta) * w) * CHUNK
            sbuf = sbufs[j * N_STAGE + s]
            sbuf[:, :] = work[pl.ds(send_lo, L),
                              j * CD:(j + 1) * CD].astype(jnp.bfloat16)
            rdmas[j] = pltpu.make_async_remote_copy(
                src_ref=sbuf,
                dst_ref=rbufs[j * N_STAGE + s],
                send_sem=rs_ssem.at[j * N_STAGE + s],
                recv_sem=rs_rsem.at[j * N_STAGE + s],
                device_id=(partners[ax],),
                device_id_type=pl.DeviceIdType.MESH,
            )
            rdmas[j].start()
        for j, order in enumerate(LANE_ORDERS):
            ax = order[s]
            beta = bits[ax]
            lo = (c_lane[j] // (2 * w)) * (2 * w)
            keep_lo = (lo + beta * w) * CHUNK
            rdmas[j].wait()
            csl = slice(j * CD, (j + 1) * CD)
            work[pl.ds(keep_lo, L), csl] = (
                work[pl.ds(keep_lo, L), csl]
                + rbufs[j * N_STAGE + s][:, :].astype(jnp.float32))

    for j in range(N_LANE):
        csl = slice(j * CD, (j + 1) * CD)
        out_ref[pl.ds(c_lane[j] * CHUNK, CHUNK), csl] = (
            work[pl.ds(c_lane[j] * CHUNK, CHUNK), csl].astype(jnp.bfloat16))

    for s in range(N_STAGE):
        w = 1 << s
        L = w * CHUNK
        for j, order in enumerate(LANE_ORDERS):
            ax = order[N_STAGE - 1 - s]
            cur_lo = (c_lane[j] // w) * w * CHUNK
            rdmas[j] = pltpu.make_async_remote_copy(
                src_ref=out_ref.at[pl.ds(cur_lo, L),
                                   pl.ds(j * CD, CD)],
                dst_ref=out_ref.at[pl.ds(cur_lo, L),
                                   pl.ds(j * CD, CD)],
                send_sem=ag_ssem.at[j * N_STAGE + s],
                recv_sem=ag_rsem.at[j * N_STAGE + s],
                device_id=(partners[ax],),
                device_id_type=pl.DeviceIdType.MESH,
            )
            rdmas[j].start()
        for j in range(N_LANE):
            rdmas[j].wait()

    @functools.partial(pl.run_scoped, exit_sem=pltpu.SemaphoreType.REGULAR)
    def _(exit_sem):
        for ax in AXES:
            pl.semaphore_signal(exit_sem, inc=1, device_id=(partners[ax],),
                                device_id_type=pl.DeviceIdType.MESH)
        pl.semaphore_wait(exit_sem, len(AXES))


def kernel(x, Wq, Wo, Wk, Wv):
    xf = x.reshape(ROWS, D).astype(jnp.bfloat16)
    wq = Wq.astype(jnp.bfloat16)
    wk = Wk.astype(jnp.bfloat16)
    wv = Wv.astype(jnp.bfloat16)
    wo = Wo.astype(jnp.bfloat16)

    comm_bufs = [
        pltpu.VMEM(((16 >> s) * CHUNK, CD), jnp.bfloat16)
        for _ in range(N_LANE) for s in range(N_STAGE)
    ]
    out = pl.pallas_call(
        _body,
        out_shape=jax.ShapeDtypeStruct((ROWS, D), jnp.bfloat16),
        in_specs=[pl.BlockSpec(memory_space=pltpu.VMEM)] * 5,
        out_specs=pl.BlockSpec(memory_space=pltpu.VMEM),
        scratch_shapes=(
            [pltpu.VMEM((ROWS, D), jnp.float32)]
            + comm_bufs
            + comm_bufs
            + [pltpu.SemaphoreType.DMA((N_LANE * N_STAGE,))] * 4
        ),
        compiler_params=pltpu.CompilerParams(collective_id=0),
    )(xf, wq, wo, wk, wv)
    return out.astype(jnp.float32).reshape(B, SQ, D)
